# baseline (speedup 1.0000x reference)
import functools

import jax
import jax.numpy as jnp
from jax import lax
from jax.experimental import pallas as pl
from jax.experimental.pallas import tpu as pltpu
from jax.experimental.pallas import tpu_sc as plsc

_BN = 32768
_H = _BN // 2


def _transpose_kernel(x_ref, o_ref):
    x = x_ref[...]
    xx = jnp.concatenate([x[:, :_H], x[:, _H:]], axis=0)
    o_ref[...] = xx.T


def _transpose_table(params_t, V, D):
    grid = pl.cdiv(V, _BN)
    return pl.pallas_call(
        _transpose_kernel,
        grid=(grid,),
        in_specs=[pl.BlockSpec((D, _BN), lambda b: (0, b))],
        out_specs=pl.BlockSpec((_H, 2 * D), lambda b: (b, 0)),
        out_shape=jax.ShapeDtypeStruct((grid * _H, 2 * D), jnp.float32),
        compiler_params=pltpu.CompilerParams(
            vmem_limit_bytes=100 * 1024 * 1024
        ),
    )(params_t)


def _emb_kernel(B4, S, D, b_per_w, C, n_chunks):
    mesh = plsc.VectorSubcoreMesh(core_axis_name="c", subcore_axis_name="s")
    HC = C // 2
    HB = HC // S
    CB = C // S

    @functools.partial(
        pl.kernel,
        mesh=mesh,
        out_type=jax.ShapeDtypeStruct((S, D, B4), jnp.float32),
        scratch_types=[
            pltpu.VMEM((b_per_w,), jnp.int32),
            pltpu.VMEM((HC, D), jnp.float32),
            pltpu.VMEM((HC, D), jnp.float32),
            pltpu.VMEM((S, D, CB), jnp.float32),
            pltpu.SemaphoreType.DMA,
            pltpu.SemaphoreType.DMA,
            pltpu.SemaphoreType.DMA,
        ],
        compiler_params=pltpu.CompilerParams(
            use_tc_tiling_on_sc=False, needs_layout_passes=False
        ),
    )
    def emb(table_hbm, idx_hbm, out_hbm, idx_v, rows0, rows1, staged, g0, g1, so):
        nc = 2
        wid = lax.axis_index("s") * nc + lax.axis_index("c")
        base = wid * b_per_w
        pltpu.sync_copy(idx_hbm.at[pl.ds(base, b_per_w)], idx_v)

        def remap(k, _):
            v = idx_v[pl.ds(k * 16, 16)]
            j = (
                (v & jnp.int32(~(_BN - 1)))
                | ((v & jnp.int32(_H - 1)) << 1)
                | ((v >> (_H.bit_length() - 1)) & jnp.int32(1))
            )
            idx_v[pl.ds(k * 16, 16)] = j
            return 0

        lax.fori_loop(0, b_per_w // 16, remap, 0)

        bufs = (rows0, rows1)
        gsems = (g0, g1)
        iota16 = lax.iota(jnp.int32, 16)
        n_halves = 2 * n_chunks

        def gather(hh):
            b = hh % 2
            return pltpu.async_copy(
                table_hbm.at[idx_v.at[pl.ds(hh * HC, HC)]], bufs[b], gsems[b]
            )

        def shuffle(hh):
            h = hh % 2
            rows = bufs[hh % 2]

            SG = 10

            def body(bb, _):
                b_dst = jnp.zeros((16,), jnp.int32) + (h * HB + bb)

                def sgroup(sg, _):
                    s0 = sg * SG
                    for ss in range(SG):
                        r_src = jnp.zeros((16,), jnp.int32) + (
                            bb * S + s0 + ss
                        )
                        s_dst = jnp.zeros((16,), jnp.int32) + (s0 + ss)
                        for k in range(D // 16):
                            d_vec = k * 16 + iota16
                            vec = plsc.load_gather(rows, [r_src, d_vec])
                            plsc.store_scatter(
                                staged, [s_dst, d_vec, b_dst], vec
                            )
                    return 0

                lax.fori_loop(0, S // SG, sgroup, 0)
                return 0

            lax.fori_loop(0, HB, body, 0)

        def out_dma(ci):
            b0 = wid * (b_per_w // S) + ci * CB
            return pltpu.async_copy(
                staged, out_hbm.at[:, :, pl.ds(b0, CB)], so
            )

        outs = [None] * n_chunks
        g = [None] * n_halves
        g[0] = gather(0)
        for hh in range(n_halves):
            if hh + 1 < n_halves:
                g[hh + 1] = gather(hh + 1)
            g[hh].wait()
            ci, h = hh // 2, hh % 2
            if h == 0 and ci > 0:
                outs[ci - 1].wait()
            shuffle(hh)
            if h == 1:
                outs[ci] = out_dma(ci)
        outs[n_chunks - 1].wait()

    return emb


def kernel(params, ids):
    V, D = params.shape
    ids_shape = ids.shape
    B4, S = ids_shape
    B = B4 * S
    NW = 32
    b_per_w = B // NW
    C = 16 * S
    n_chunks = b_per_w // C

    packed = _transpose_table(params.T, V, D)
    table = packed.reshape(packed.shape[0] * 2, D)
    ids_flat = ids.reshape((B,)).astype(jnp.int32)
    out = _emb_kernel(B4, S, D, b_per_w, C, n_chunks)(table, ids_flat)
    return out.transpose(2, 0, 1)


# speedup vs baseline: 1.1444x; 1.1444x over previous
import functools

import jax
import jax.numpy as jnp
from jax import lax
from jax.experimental import pallas as pl
from jax.experimental.pallas import tpu as pltpu
from jax.experimental.pallas import tpu_sc as plsc

_BN = 32768
_H = _BN // 2


def _transpose_kernel(x_ref, o_ref):
    x = x_ref[...]
    xx = jnp.concatenate([x[:, :_H], x[:, _H:]], axis=0)
    o_ref[...] = xx.T


def _transpose_table(params_t, V, D):
    grid = pl.cdiv(V, _BN)
    return pl.pallas_call(
        _transpose_kernel,
        grid=(grid,),
        in_specs=[pl.BlockSpec((D, _BN), lambda b: (0, b))],
        out_specs=pl.BlockSpec((_H, 2 * D), lambda b: (b, 0)),
        out_shape=jax.ShapeDtypeStruct((grid * _H, 2 * D), jnp.float32),
        compiler_params=pltpu.CompilerParams(
            vmem_limit_bytes=100 * 1024 * 1024
        ),
    )(params_t)


def _emb_kernel(B, D, b_per_w, C, n_chunks, out_shape3):
    mesh = plsc.VectorSubcoreMesh(core_axis_name="c", subcore_axis_name="s")

    @functools.partial(
        pl.kernel,
        mesh=mesh,
        out_type=jax.ShapeDtypeStruct(out_shape3, jnp.float32),
        scratch_types=[
            pltpu.VMEM((b_per_w,), jnp.int32),
            pltpu.VMEM((C, D), jnp.float32),
            pltpu.VMEM((C, D), jnp.float32),
            pltpu.SemaphoreType.DMA,
            pltpu.SemaphoreType.DMA,
            pltpu.SemaphoreType.DMA,
            pltpu.SemaphoreType.DMA,
        ],
        compiler_params=pltpu.CompilerParams(use_tc_tiling_on_sc=False),
    )
    def emb(table_hbm, idx_hbm, out3_hbm, idx_v, rows0, rows1, g0, g1, s0, s1):
        S = out_shape3[1]
        CB = C // S
        nc = 2
        wid = lax.axis_index("s") * nc + lax.axis_index("c")
        base = wid * b_per_w
        pltpu.sync_copy(idx_hbm.at[pl.ds(base, b_per_w)], idx_v)

        def remap(k, _):
            v = idx_v[pl.ds(k * 16, 16)]
            j = (
                (v & jnp.int32(~(_BN - 1)))
                | ((v & jnp.int32(_H - 1)) << 1)
                | ((v >> (_H.bit_length() - 1)) & jnp.int32(1))
            )
            idx_v[pl.ds(k * 16, 16)] = j
            return 0

        lax.fori_loop(0, b_per_w // 16, remap, 0)

        bufs = (rows0, rows1)
        gsems = (g0, g1)
        ssems = (s0, s1)

        def gather(ci):
            b = ci % 2
            return pltpu.async_copy(
                table_hbm.at[idx_v.at[pl.ds(ci * C, C)]], bufs[b], gsems[b]
            )

        def scatter(ci):
            b = ci % 2
            gb0 = (base + ci * C) // S
            handles = []
            for bb in range(CB):
                handles.append(
                    pltpu.async_copy(
                        bufs[b].at[pl.ds(bb * S, S)],
                        out3_hbm.at[gb0 + bb],
                        ssems[b],
                    )
                )
            return handles

        gathers = [None] * n_chunks
        scatters = [None] * n_chunks
        gathers[0] = gather(0)
        for ci in range(n_chunks):
            if ci + 1 < n_chunks:
                if ci - 1 >= 0:
                    for h in scatters[ci - 1]:
                        h.wait()
                gathers[ci + 1] = gather(ci + 1)
            gathers[ci].wait()
            scatters[ci] = scatter(ci)
        for h in scatters[n_chunks - 2]:
            h.wait()
        for h in scatters[n_chunks - 1]:
            h.wait()

    return emb


def kernel(params, ids):
    V, D = params.shape
    ids_shape = ids.shape
    B = 1
    for s in ids_shape:
        B *= s
    NW = 32
    b_per_w = B // NW
    C = 800
    n_chunks = b_per_w // C

    packed = _transpose_table(params.T, V, D)
    table = packed.reshape(packed.shape[0] * 2, D)
    ids_flat = ids.reshape((B,)).astype(jnp.int32)
    out = _emb_kernel(
        B, D, b_per_w, C, n_chunks, tuple(ids_shape) + (D,)
    )(table, ids_flat)
    return out


# speedup vs baseline: 1.1494x; 1.0044x over previous
import functools

import jax
import jax.numpy as jnp
from jax import lax
from jax.experimental import pallas as pl
from jax.experimental.pallas import tpu as pltpu
from jax.experimental.pallas import tpu_sc as plsc

_BN = 32768
_H = _BN // 2


def _transpose_kernel(x_ref, o_ref):
    x = x_ref[...]
    xx = jnp.concatenate([x[:, :_H], x[:, _H:]], axis=0)
    o_ref[...] = xx.T


def _transpose_table(params_t, V, D):
    grid = pl.cdiv(V, _BN)
    return pl.pallas_call(
        _transpose_kernel,
        grid=(grid,),
        in_specs=[pl.BlockSpec((D, _BN), lambda b: (0, b))],
        out_specs=pl.BlockSpec((_H, 2 * D), lambda b: (b, 0)),
        out_shape=jax.ShapeDtypeStruct((grid * _H, 2 * D), jnp.float32),
        compiler_params=pltpu.CompilerParams(
            vmem_limit_bytes=100 * 1024 * 1024
        ),
    )(params_t)


def _emb_kernel(B, D, b_per_w, C, n_chunks):
    mesh = plsc.VectorSubcoreMesh(core_axis_name="c", subcore_axis_name="s")

    @functools.partial(
        pl.kernel,
        mesh=mesh,
        out_type=jax.ShapeDtypeStruct((B, D), jnp.float32),
        scratch_types=[
            pltpu.VMEM((b_per_w,), jnp.int32),
            pltpu.VMEM((C, D), jnp.float32),
            pltpu.VMEM((C, D), jnp.float32),
            pltpu.SemaphoreType.DMA,
            pltpu.SemaphoreType.DMA,
            pltpu.SemaphoreType.DMA,
            pltpu.SemaphoreType.DMA,
        ],
        compiler_params=pltpu.CompilerParams(use_tc_tiling_on_sc=False),
    )
    def emb(table_hbm, idx_hbm, out_hbm, idx_v, rows0, rows1, g0, g1, s0, s1):
        nc = 2
        wid = lax.axis_index("s") * nc + lax.axis_index("c")
        base = wid * b_per_w
        pltpu.sync_copy(idx_hbm.at[pl.ds(base, b_per_w)], idx_v)

        def remap(k, _):
            v = idx_v[pl.ds(k * 16, 16)]
            j = (
                (v & jnp.int32(~(_BN - 1)))
                | ((v & jnp.int32(_H - 1)) << 1)
                | ((v >> (_H.bit_length() - 1)) & jnp.int32(1))
            )
            idx_v[pl.ds(k * 16, 16)] = j
            return 0

        lax.fori_loop(0, b_per_w // 16, remap, 0)

        bufs = (rows0, rows1)
        gsems = (g0, g1)
        ssems = (s0, s1)

        def gather(ci):
            b = ci % 2
            return pltpu.async_copy(
                table_hbm.at[idx_v.at[pl.ds(ci * C, C)]], bufs[b], gsems[b]
            )

        def scatter(ci):
            b = ci % 2
            return pltpu.async_copy(
                bufs[b], out_hbm.at[pl.ds(base + ci * C, C)], ssems[b]
            )

        gathers = [None] * n_chunks
        scatters = [None] * n_chunks
        gathers[0] = gather(0)
        for ci in range(n_chunks):
            if ci + 1 < n_chunks:
                if ci - 1 >= 0:
                    scatters[ci - 1].wait()
                gathers[ci + 1] = gather(ci + 1)
            gathers[ci].wait()
            scatters[ci] = scatter(ci)
        scatters[n_chunks - 2].wait()
        scatters[n_chunks - 1].wait()

    return emb


def kernel(params, ids):
    V, D = params.shape
    ids_shape = ids.shape
    B = 1
    for s in ids_shape:
        B *= s
    NW = 32
    b_per_w = B // NW
    C = 800
    n_chunks = b_per_w // C

    packed = _transpose_table(params.T, V, D)
    table = packed.reshape(packed.shape[0] * 2, D)
    ids_flat = ids.reshape((B,)).astype(jnp.int32)
    out = _emb_kernel(B, D, b_per_w, C, n_chunks)(table, ids_flat)
    return out.reshape(tuple(ids_shape) + (D,))
